# baseline (speedup 1.0000x reference)
import jax
import jax.numpy as jnp
from jax import lax
from jax.experimental import pallas as pl

_B, _T, _A, _D = 8, 512, 32, 256
_NZ, _NQ, _NL = 128, 64, 20

_PREC = lax.Precision.HIGHEST


def _mm(a, b):
    return lax.dot_general(a, b, (((1,), (0,)), ((), ())),
                           precision=_PREC,
                           preferred_element_type=jnp.float32)


def _conv_stack(x, w_ref, b_ref):
    for s in range(2):
        for i in range(10):
            idx = s * 10 + i
            d = 2 ** i
            y = _mm(x, w_ref[idx, 1])
            if d < _T:
                xm = jnp.concatenate(
                    [jnp.zeros((d, _D), jnp.float32), x[: _T - d]], axis=0)
                xp = jnp.concatenate(
                    [x[d:], jnp.zeros((d, _D), jnp.float32)], axis=0)
                y = y + _mm(xm, w_ref[idx, 0]) + _mm(xp, w_ref[idx, 2])
            y = y + b_ref[idx][None, :]
            x = jnp.maximum(y, 0.0) + x
    return x


def _vqvae_kernel(acts_ref, epw_ref, epb_ref, ecw_ref, ecb_ref, g_ref, lb_ref,
                  cbz_ref, cbzT_ref, cbq_ref, cbqT_ref, dcw_ref, dcb_ref,
                  dpw_ref, dpb_ref, out_ref):
    def body(b, carry):
        a = acts_ref[b]
        x = _mm(a, epw_ref[...]) + epb_ref[0][None, :]
        x = _conv_stack(x, ecw_ref, ecb_ref)
        mu = jnp.mean(x, axis=-1, keepdims=True)
        xc = x - mu
        var = jnp.mean(xc * xc, axis=-1, keepdims=True)
        xln = xc / jnp.sqrt(var + 1e-5) * g_ref[0][None, :] + lb_ref[0][None, :]
        sim_z = _mm(xln, cbzT_ref[...])
        iz = jnp.argmax(sim_z, axis=-1)
        oh_z = (lax.broadcasted_iota(jnp.int32, (_T, _NZ), 1)
                == iz[:, None]).astype(jnp.float32)
        zq = _mm(oh_z, cbz_ref[...])
        sim_q = _mm(zq, cbqT_ref[...])
        iq = jnp.argmax(sim_q, axis=-1)
        oh_q = (lax.broadcasted_iota(jnp.int32, (_T, _NQ), 1)
                == iq[:, None]).astype(jnp.float32)
        qq = _mm(oh_q, cbq_ref[...])
        y = _conv_stack(qq, dcw_ref, dcb_ref)
        out_ref[b] = _mm(y, dpw_ref[...]) + dpb_ref[0][None, :]
        return carry

    lax.fori_loop(0, _B, body, 0)


def kernel(actions, enc_proj_w, enc_proj_b, enc_conv_w, enc_conv_b, ln_g, ln_b,
           codebook_z, codebook_q, dec_conv_w, dec_conv_b, dec_proj_w,
           dec_proj_b):
    ecw = jnp.transpose(enc_conv_w, (0, 3, 2, 1))
    dcw = jnp.transpose(dec_conv_w, (0, 3, 2, 1))
    out = pl.pallas_call(
        _vqvae_kernel,
        out_shape=jax.ShapeDtypeStruct((_B, _T, _A), jnp.float32),
    )(actions, enc_proj_w, enc_proj_b.reshape(1, _D), ecw, enc_conv_b,
      ln_g.reshape(1, _D), ln_b.reshape(1, _D),
      codebook_z, codebook_z.T, codebook_q, codebook_q.T,
      dcw, dec_conv_b, dec_proj_w, dec_proj_b.reshape(1, _A))
    return out


# speedup vs baseline: 1.7579x; 1.7579x over previous
import jax
import jax.numpy as jnp
from jax import lax
from jax.experimental import pallas as pl

_B, _T, _A, _D = 8, 512, 32, 256
_NZ, _NQ, _NL = 128, 64, 20


def _bmm(a, b):
    return lax.dot_general(a, b, (((1,), (0,)), ((), ())),
                           preferred_element_type=jnp.float32)


def _split(x):
    hi = x.astype(jnp.bfloat16)
    lo = (x - hi.astype(jnp.float32)).astype(jnp.bfloat16)
    return hi, lo


def _mm3(a, bh, bl):
    ah, al = _split(a)
    return _bmm(ah, bh) + (_bmm(ah, bl) + _bmm(al, bh))


def _conv_stack(x, wh_ref, wl_ref, b_ref):
    for s in range(2):
        for i in range(10):
            idx = s * 10 + i
            d = 2 ** i
            xh, xl = _split(x)
            y = (_bmm(xh, wh_ref[idx, 1])
                 + (_bmm(xh, wl_ref[idx, 1]) + _bmm(xl, wh_ref[idx, 1])))
            if d < _T:
                zh = jnp.zeros((d, _D), jnp.bfloat16)
                xmh = jnp.concatenate([zh, xh[: _T - d]], axis=0)
                xml = jnp.concatenate([zh, xl[: _T - d]], axis=0)
                xph = jnp.concatenate([xh[d:], zh], axis=0)
                xpl = jnp.concatenate([xl[d:], zh], axis=0)
                y = y + (_bmm(xmh, wh_ref[idx, 0])
                         + (_bmm(xmh, wl_ref[idx, 0])
                            + _bmm(xml, wh_ref[idx, 0])))
                y = y + (_bmm(xph, wh_ref[idx, 2])
                         + (_bmm(xph, wl_ref[idx, 2])
                            + _bmm(xpl, wh_ref[idx, 2])))
            y = y + b_ref[idx][None, :]
            x = jnp.maximum(y, 0.0) + x
    return x


def _vqvae_kernel(acts_ref, epwh_ref, epwl_ref, epb_ref,
                  ecwh_ref, ecwl_ref, ecb_ref, g_ref, lb_ref,
                  cbz_ref, cbzTh_ref, cbzTl_ref, cbq_ref, cbqTh_ref,
                  cbqTl_ref, dcwh_ref, dcwl_ref, dcb_ref,
                  dpwh_ref, dpwl_ref, dpb_ref, out_ref):
    def body(b, carry):
        a = acts_ref[b]
        x = _mm3(a, epwh_ref[...], epwl_ref[...]) + epb_ref[0][None, :]
        x = _conv_stack(x, ecwh_ref, ecwl_ref, ecb_ref)
        mu = jnp.mean(x, axis=-1, keepdims=True)
        xc = x - mu
        var = jnp.mean(xc * xc, axis=-1, keepdims=True)
        xln = xc / jnp.sqrt(var + 1e-5) * g_ref[0][None, :] + lb_ref[0][None, :]
        sim_z = _mm3(xln, cbzTh_ref[...], cbzTl_ref[...])
        iz = jnp.argmax(sim_z, axis=-1)
        oh_z = (lax.broadcasted_iota(jnp.int32, (_T, _NZ), 1)
                == iz[:, None]).astype(jnp.float32)
        zq = lax.dot_general(oh_z, cbz_ref[...], (((1,), (0,)), ((), ())),
                             precision=lax.Precision.HIGHEST,
                             preferred_element_type=jnp.float32)
        sim_q = _mm3(zq, cbqTh_ref[...], cbqTl_ref[...])
        iq = jnp.argmax(sim_q, axis=-1)
        oh_q = (lax.broadcasted_iota(jnp.int32, (_T, _NQ), 1)
                == iq[:, None]).astype(jnp.float32)
        qq = lax.dot_general(oh_q, cbq_ref[...], (((1,), (0,)), ((), ())),
                             precision=lax.Precision.HIGHEST,
                             preferred_element_type=jnp.float32)
        y = _conv_stack(qq, dcwh_ref, dcwl_ref, dcb_ref)
        out_ref[b] = (_mm3(y, dpwh_ref[...], dpwl_ref[...])
                      + dpb_ref[0][None, :])
        return carry

    lax.fori_loop(0, _B, body, 0)


def kernel(actions, enc_proj_w, enc_proj_b, enc_conv_w, enc_conv_b, ln_g, ln_b,
           codebook_z, codebook_q, dec_conv_w, dec_conv_b, dec_proj_w,
           dec_proj_b):
    def split(x):
        hi = x.astype(jnp.bfloat16)
        lo = (x - hi.astype(jnp.float32)).astype(jnp.bfloat16)
        return hi, lo

    ecwh, ecwl = split(jnp.transpose(enc_conv_w, (0, 3, 2, 1)))
    dcwh, dcwl = split(jnp.transpose(dec_conv_w, (0, 3, 2, 1)))
    epwh, epwl = split(enc_proj_w)
    dpwh, dpwl = split(dec_proj_w)
    cbzTh, cbzTl = split(codebook_z.T)
    cbqTh, cbqTl = split(codebook_q.T)
    out = pl.pallas_call(
        _vqvae_kernel,
        out_shape=jax.ShapeDtypeStruct((_B, _T, _A), jnp.float32),
    )(actions, epwh, epwl, enc_proj_b.reshape(1, _D), ecwh, ecwl, enc_conv_b,
      ln_g.reshape(1, _D), ln_b.reshape(1, _D),
      codebook_z, cbzTh, cbzTl, codebook_q, cbqTh, cbqTl,
      dcwh, dcwl, dec_conv_b, dpwh, dpwl, dec_proj_b.reshape(1, _A))
    return out
